# baseline (speedup 1.0000x reference)
import functools

import jax
import jax.numpy as jnp
from jax import lax
from jax.experimental import pallas as pl
from jax.experimental.pallas import tpu as pltpu
from jax.experimental.pallas import tpu_sc as plsc

B = 16384
D = 300
NC = 2
NS = 16
NW = NC * NS
B_PER_W = B // NW
CHUNK = 128
NCHUNK = B_PER_W // CHUNK


def _sc_gather(src_table, tgt_table, src_id3, tgt_id3):
  mesh = plsc.VectorSubcoreMesh(core_axis_name="c", subcore_axis_name="s")

  @functools.partial(
      pl.kernel,
      mesh=mesh,
      out_type=jax.ShapeDtypeStruct((2 * B, D), jnp.float32),
      scratch_types=[
          pltpu.VMEM((NCHUNK, CHUNK), jnp.int32),
          pltpu.VMEM((NCHUNK, CHUNK), jnp.int32),
          pltpu.VMEM((CHUNK, D), jnp.float32),
          pltpu.VMEM((CHUNK, D), jnp.float32),
          pltpu.SemaphoreType.DMA,
          pltpu.SemaphoreType.DMA,
      ],
  )
  def gather(src_hbm, tgt_hbm, sid_hbm, tid_hbm, out_hbm,
             sidx, tidx, buf0, buf1, sem0, sem1):
    wid = lax.axis_index("s") * NC + lax.axis_index("c")
    base = wid * B_PER_W
    pltpu.sync_copy(sid_hbm.at[wid], sidx)
    pltpu.sync_copy(tid_hbm.at[wid], tidx)

    bufs = (buf0, buf1)
    sems = (sem0, sem1)

    def chunk(i):
      if i < NCHUNK:
        return src_hbm, sidx.at[i], base + i * CHUNK
      j = i - NCHUNK
      return tgt_hbm, tidx.at[j], B + base + j * CHUNK

    total = 2 * NCHUNK
    handles = [None] * total
    for i in range(min(2, total)):
      tab, idx, _ = chunk(i)
      handles[i] = pltpu.async_copy(tab.at[idx], bufs[i % 2], sems[i % 2])
    for i in range(total):
      _, _, out_base = chunk(i)
      handles[i].wait()
      pltpu.sync_copy(bufs[i % 2], out_hbm.at[pl.ds(out_base, CHUNK)])
      if i + 2 < total:
        tab, idx, _ = chunk(i + 2)
        handles[i + 2] = pltpu.async_copy(tab.at[idx], bufs[i % 2], sems[i % 2])

  return gather(src_table, tgt_table, src_id3, tgt_id3)


def _tc_map(gathered, W_enc, b_enc2, W_dec):
  BM = 2048

  def body(g_ref, we_ref, be_ref, wd_ref, out_ref, w_scr, b_scr):
    @pl.when(pl.program_id(0) == 0)
    def _():
      w_scr[...] = lax.dot_general(
          we_ref[...], wd_ref[...], (((0,), (0,)), ((), ())),
          preferred_element_type=jnp.float32)
      b_scr[...] = lax.dot_general(
          be_ref[...], wd_ref[...], (((1,), (0,)), ((), ())),
          preferred_element_type=jnp.float32)
    out_ref[...] = lax.dot_general(
        g_ref[...], w_scr[...], (((1,), (0,)), ((), ())),
        preferred_element_type=jnp.float32) + b_scr[...]

  return pl.pallas_call(
      body,
      grid=(B // BM,),
      in_specs=[
          pl.BlockSpec((BM, D), lambda i: (i, 0)),
          pl.BlockSpec((D, D), lambda i: (0, 0)),
          pl.BlockSpec((1, D), lambda i: (0, 0)),
          pl.BlockSpec((D, D), lambda i: (0, 0)),
      ],
      out_specs=pl.BlockSpec((BM, D), lambda i: (i, 0)),
      out_shape=jax.ShapeDtypeStruct((2 * B, D), jnp.float32),
      scratch_shapes=[
          pltpu.VMEM((D, D), jnp.float32),
          pltpu.VMEM((1, D), jnp.float32),
      ],
      input_output_aliases={0: 0},
  )(gathered, W_enc, b_enc2, W_dec)


def kernel(src_table, tgt_table, W_enc, b_enc, W_dec, src_id, tgt_id):
  gathered = jnp.concatenate([
      jnp.take(src_table, src_id, axis=0),
      jnp.take(tgt_table, tgt_id, axis=0)], axis=0)
  return _tc_map(gathered, W_enc, b_enc.reshape(1, D), W_dec)
